# baseline (speedup 1.0000x reference)
import jax
import jax.numpy as jnp
from jax import lax
from jax.experimental import pallas as pl
from jax.experimental.pallas import tpu as pltpu
from jax.experimental.pallas import tpu_sc as plsc

_ROWS = 100000
_DIM = 128
_CHUNK = 400
_NCHUNKS = _ROWS // _CHUNK
_NC = 2
_NW = 32
_FULL = _NCHUNKS // _NW
_REM = _NCHUNKS - _FULL * _NW


def _sc_body(in_hbm, out_hbm, buf):
    wid = lax.axis_index("s") * _NC + lax.axis_index("c")

    def do_chunk(k):
        c = wid + k * _NW
        pltpu.sync_copy(in_hbm.at[pl.ds(c * _CHUNK, _CHUNK)], buf)
        pltpu.sync_copy(buf, out_hbm.at[pl.ds(c * _CHUNK, _CHUNK)])

    for k in range(_FULL):
        do_chunk(k)

    @pl.when(wid < _REM)
    def _():
        do_chunk(_FULL)


def kernel(weight):
    mesh = plsc.VectorSubcoreMesh(core_axis_name="c", subcore_axis_name="s")
    f = pl.kernel(
        _sc_body,
        mesh=mesh,
        out_type=jax.ShapeDtypeStruct((_ROWS, _DIM), jnp.float32),
        scratch_types=[pltpu.VMEM((_CHUNK, _DIM), jnp.float32)],
    )
    return f(weight)


# speedup vs baseline: 1.0605x; 1.0605x over previous
import jax
import jax.numpy as jnp
from jax import lax
from jax.experimental import pallas as pl
from jax.experimental.pallas import tpu as pltpu
from jax.experimental.pallas import tpu_sc as plsc

_ROWS = 100000
_DIM = 128
_CHUNK = 400
_NCHUNKS = _ROWS // _CHUNK
_NC = 2
_NW = 32
_FULL = _NCHUNKS // _NW
_REM = _NCHUNKS - _FULL * _NW


def _sc_body(in_hbm, out_hbm, buf0, buf1, isem0, isem1, osem0, osem1):
    wid = lax.axis_index("s") * _NC + lax.axis_index("c")
    bufs, isems, osems = (buf0, buf1), (isem0, isem1), (osem0, osem1)

    def src(k):
        return in_hbm.at[pl.ds((wid + k * _NW) * _CHUNK, _CHUNK)]

    def dst(k):
        return out_hbm.at[pl.ds((wid + k * _NW) * _CHUNK, _CHUNK)]

    def start_in(k):
        pltpu.async_copy(src(k), bufs[k % 2], isems[k % 2])

    def wait_in(k):
        pltpu.make_async_copy(src(k), bufs[k % 2], isems[k % 2]).wait()

    def start_out(k):
        pltpu.async_copy(bufs[k % 2], dst(k), osems[k % 2])

    def wait_out(k):
        pltpu.make_async_copy(bufs[k % 2], dst(k), osems[k % 2]).wait()

    start_in(0)
    for k in range(_FULL):
        wait_in(k)
        start_out(k)
        if k >= 1:
            wait_out(k - 1)
        if k + 1 < _FULL:
            start_in(k + 1)
        else:
            @pl.when(wid < _REM)
            def _():
                start_in(_FULL)

    @pl.when(wid < _REM)
    def _():
        wait_in(_FULL)
        start_out(_FULL)
        wait_out(_FULL - 1)
        wait_out(_FULL)

    @pl.when(wid >= _REM)
    def _():
        wait_out(_FULL - 1)


def kernel(weight):
    mesh = plsc.VectorSubcoreMesh(core_axis_name="c", subcore_axis_name="s")
    f = pl.kernel(
        _sc_body,
        mesh=mesh,
        out_type=jax.ShapeDtypeStruct((_ROWS, _DIM), jnp.float32),
        scratch_types=[
            pltpu.VMEM((_CHUNK, _DIM), jnp.float32),
            pltpu.VMEM((_CHUNK, _DIM), jnp.float32),
            pltpu.SemaphoreType.DMA,
            pltpu.SemaphoreType.DMA,
            pltpu.SemaphoreType.DMA,
            pltpu.SemaphoreType.DMA,
        ],
    )
    return f(weight)


# speedup vs baseline: 1.8348x; 1.7300x over previous
import jax
import jax.numpy as jnp
from jax.experimental import pallas as pl
from jax.experimental.pallas import tpu as pltpu

_ROWS = 100000
_DIM = 128
_CHUNK_ROWS = 5000
_NCHUNKS = _ROWS // _CHUNK_ROWS


def _copy_body(in_hbm, out_hbm, bufs, in_sems, out_sems):
    def copy_in(c):
        return pltpu.make_async_copy(
            in_hbm.at[pl.ds(c * _CHUNK_ROWS, _CHUNK_ROWS)], bufs.at[c],
            in_sems.at[c])

    def copy_out(c):
        return pltpu.make_async_copy(
            bufs.at[c], out_hbm.at[pl.ds(c * _CHUNK_ROWS, _CHUNK_ROWS)],
            out_sems.at[c])

    for c in range(_NCHUNKS):
        copy_in(c).start()
    for c in range(_NCHUNKS):
        copy_in(c).wait()
        copy_out(c).start()
    for c in range(_NCHUNKS):
        copy_out(c).wait()


def kernel(weight):
    return pl.pallas_call(
        _copy_body,
        in_specs=[pl.BlockSpec(memory_space=pl.ANY)],
        out_specs=pl.BlockSpec(memory_space=pl.ANY),
        out_shape=jax.ShapeDtypeStruct((_ROWS, _DIM), jnp.float32),
        scratch_shapes=[
            pltpu.VMEM((_NCHUNKS, _CHUNK_ROWS, _DIM), jnp.float32),
            pltpu.SemaphoreType.DMA((_NCHUNKS,)),
            pltpu.SemaphoreType.DMA((_NCHUNKS,)),
        ],
    )(weight)
